# baseline (speedup 1.0000x reference)
import functools

import jax
import jax.numpy as jnp
from jax import lax
from jax.experimental import pallas as pl
from jax.experimental.pallas import tpu as pltpu
from jax.experimental.pallas import tpu_sc as plsc

_CHUNK_ROWS = 16
_NSLOT = 5


def _make_sc_add(B, S, D, NC, NS, L):
    NW = NC * NS
    rows_per_tile = S // NW
    chunk_rows = _CHUNK_ROWS
    n_chunks = rows_per_tile // chunk_rows
    NSLOT = _NSLOT
    steps = [(c, b) for c in range(n_chunks) for b in range(B)]
    n_steps = len(steps)
    lookahead = NSLOT - 1

    mesh = plsc.VectorSubcoreMesh(core_axis_name="c", subcore_axis_name="s")

    scratch = (
        [pltpu.VMEM((chunk_rows, D), jnp.float32)] * (NSLOT + 2)
        + [pltpu.SemaphoreType.DMA] * (2 * NSLOT + 2)
    )

    @functools.partial(
        pl.kernel,
        mesh=mesh,
        out_type=jax.ShapeDtypeStruct((B * S, D), jnp.float32),
        scratch_types=scratch,
    )
    def sc_add(x_hbm, emb_hbm, out_hbm, *refs):
        xbuf = list(refs[:NSLOT])
        ebuf = list(refs[NSLOT:NSLOT + 2])
        insem = list(refs[NSLOT + 2:2 * NSLOT + 2])
        outsem = list(refs[2 * NSLOT + 2:3 * NSLOT + 2])
        esem = list(refs[3 * NSLOT + 2:3 * NSLOT + 4])
        wid = lax.axis_index("s") * NC + lax.axis_index("c")
        base = wid * rows_per_tile

        def row0(c, b):
            return b * S + base + c * chunk_rows

        def in_copy(i):
            c, b = steps[i]
            slot = i % NSLOT
            return pltpu.make_async_copy(
                x_hbm.at[pl.ds(row0(c, b), chunk_rows)], xbuf[slot], insem[slot]
            )

        def out_copy(i):
            c, b = steps[i]
            slot = i % NSLOT
            return pltpu.make_async_copy(
                xbuf[slot], out_hbm.at[pl.ds(row0(c, b), chunk_rows)], outsem[slot]
            )

        def e_copy(c):
            return pltpu.make_async_copy(
                emb_hbm.at[pl.ds(base + c * chunk_rows, chunk_rows)],
                ebuf[c % 2],
                esem[c % 2],
            )

        e_copy(0).start()
        for i in range(min(lookahead, n_steps)):
            in_copy(i).start()
        for i in range(n_steps):
            c, b = steps[i]
            slot = i % NSLOT
            j = i + lookahead
            if j < n_steps:
                if j >= NSLOT:
                    out_copy(j - NSLOT).wait()
                in_copy(j).start()
            if b == 0 and c + 1 < n_chunks:
                e_copy(c + 1).start()
            in_copy(i).wait()
            if b == 0:
                e_copy(c).wait()
            xb = xbuf[slot]
            eb = ebuf[c % 2]

            @plsc.parallel_loop(0, chunk_rows, 1)
            def _(r):
                @plsc.parallel_loop(0, D, L, unroll=8)
                def _(k):
                    xb[r, pl.ds(k, L)] = xb[r, pl.ds(k, L)] + eb[r, pl.ds(k, L)]

            out_copy(i).start()
        for i in range(max(0, n_steps - NSLOT), n_steps):
            out_copy(i).wait()

    return sc_add


def kernel(x, emb):
    B, S, D = x.shape
    info = plsc.get_sparse_core_info()
    NC, NS, L = info.num_cores, info.num_subcores, info.num_lanes
    sc_add = _make_sc_add(B, S, D, NC, NS, L)
    out = sc_add(x.reshape(B * S, D), emb[:S])
    return out.reshape(B, S, D)


# speedup vs baseline: 1.1592x; 1.1592x over previous
import functools

import jax
import jax.numpy as jnp
from jax import lax
from jax.experimental import pallas as pl
from jax.experimental.pallas import tpu as pltpu
from jax.experimental.pallas import tpu_sc as plsc

_CHUNK_ROWS = 8
_NSLOT = 4
_LOOKAHEAD = 2


def _make_sc_add(B, S, D, NC, NS, L):
    NW = NC * NS
    rows_per_tile = S // NW
    chunk_rows = _CHUNK_ROWS
    n_chunks = rows_per_tile // chunk_rows
    NSLOT = _NSLOT
    LA = _LOOKAHEAD

    mesh = plsc.VectorSubcoreMesh(core_axis_name="c", subcore_axis_name="s")

    scratch = (
        [pltpu.VMEM((chunk_rows, D), jnp.float32)] * (NSLOT * B + NSLOT)
        + [pltpu.SemaphoreType.DMA] * (2 * NSLOT)
    )

    @functools.partial(
        pl.kernel,
        mesh=mesh,
        out_type=jax.ShapeDtypeStruct((B * S, D), jnp.float32),
        scratch_types=scratch,
    )
    def sc_add(x_hbm, emb_hbm, out_hbm, *refs):
        xbuf = [list(refs[s * B:(s + 1) * B]) for s in range(NSLOT)]
        ebuf = list(refs[NSLOT * B:NSLOT * B + NSLOT])
        insem = list(refs[NSLOT * B + NSLOT:NSLOT * B + 2 * NSLOT])
        outsem = list(refs[NSLOT * B + 2 * NSLOT:NSLOT * B + 3 * NSLOT])
        wid = lax.axis_index("s") * NC + lax.axis_index("c")
        base = wid * rows_per_tile

        def row0(c, b):
            return b * S + base + c * chunk_rows

        def in_copies(c):
            slot = c % NSLOT
            cps = [
                pltpu.make_async_copy(
                    x_hbm.at[pl.ds(row0(c, b), chunk_rows)],
                    xbuf[slot][b],
                    insem[slot],
                )
                for b in range(B)
            ]
            cps.append(
                pltpu.make_async_copy(
                    emb_hbm.at[pl.ds(base + c * chunk_rows, chunk_rows)],
                    ebuf[slot],
                    insem[slot],
                )
            )
            return cps

        def out_copies(c):
            slot = c % NSLOT
            return [
                pltpu.make_async_copy(
                    xbuf[slot][b],
                    out_hbm.at[pl.ds(row0(c, b), chunk_rows)],
                    outsem[slot],
                )
                for b in range(B)
            ]

        for c in range(min(LA, n_chunks)):
            for cp in in_copies(c):
                cp.start()
        for c in range(n_chunks):
            slot = c % NSLOT
            j = c + LA
            if j < n_chunks:
                if j >= NSLOT:
                    for cp in out_copies(j - NSLOT):
                        cp.wait()
                for cp in in_copies(j):
                    cp.start()
            for cp in in_copies(c):
                cp.wait()
            xs = xbuf[slot]
            eb = ebuf[slot]

            @plsc.parallel_loop(0, chunk_rows, 1)
            def _(r):
                @plsc.parallel_loop(0, D, L, unroll=8)
                def _(k):
                    e = eb[r, pl.ds(k, L)]
                    for b in range(B):
                        xs[b][r, pl.ds(k, L)] = xs[b][r, pl.ds(k, L)] + e

            for cp in out_copies(c):
                cp.start()
        for c in range(max(0, n_chunks - NSLOT), n_chunks):
            for cp in out_copies(c):
                cp.wait()

    return sc_add


def kernel(x, emb):
    B, S, D = x.shape
    info = plsc.get_sparse_core_info()
    NC, NS, L = info.num_cores, info.num_subcores, info.num_lanes
    sc_add = _make_sc_add(B, S, D, NC, NS, L)
    out = sc_add(x.reshape(B * S, D), emb[:S])
    return out.reshape(B, S, D)


# speedup vs baseline: 1.1944x; 1.0304x over previous
import functools

import jax
import jax.numpy as jnp
from jax import lax
from jax.experimental import pallas as pl
from jax.experimental.pallas import tpu as pltpu
from jax.experimental.pallas import tpu_sc as plsc

_CHUNK_ROWS = 8
_NSLOT = 4
_LOOKAHEAD = 2


def _make_sc_add(B, S, D, NC, NS, L):
    NW = NC * NS
    rows_per_tile = S // NW
    chunk_rows = _CHUNK_ROWS
    n_chunks = rows_per_tile // chunk_rows
    NSLOT = _NSLOT
    LA = _LOOKAHEAD

    mesh = plsc.VectorSubcoreMesh(core_axis_name="c", subcore_axis_name="s")

    scratch = (
        [pltpu.VMEM((B, chunk_rows, D), jnp.float32)] * NSLOT
        + [pltpu.VMEM((chunk_rows, D), jnp.float32)] * NSLOT
        + [pltpu.SemaphoreType.DMA] * (2 * NSLOT)
    )

    @functools.partial(
        pl.kernel,
        mesh=mesh,
        out_type=jax.ShapeDtypeStruct((B, S, D), jnp.float32),
        scratch_types=scratch,
    )
    def sc_add(x_hbm, emb_hbm, out_hbm, *refs):
        xbuf = list(refs[:NSLOT])
        ebuf = list(refs[NSLOT:2 * NSLOT])
        insem = list(refs[2 * NSLOT:3 * NSLOT])
        outsem = list(refs[3 * NSLOT:4 * NSLOT])
        wid = lax.axis_index("s") * NC + lax.axis_index("c")
        base = wid * rows_per_tile

        def in_copies(c):
            slot = c % NSLOT
            r0 = base + c * chunk_rows
            return [
                pltpu.make_async_copy(
                    x_hbm.at[:, pl.ds(r0, chunk_rows), :],
                    xbuf[slot],
                    insem[slot],
                ),
                pltpu.make_async_copy(
                    emb_hbm.at[pl.ds(r0, chunk_rows)],
                    ebuf[slot],
                    insem[slot],
                ),
            ]

        def out_copy(c):
            slot = c % NSLOT
            r0 = base + c * chunk_rows
            return pltpu.make_async_copy(
                xbuf[slot],
                out_hbm.at[:, pl.ds(r0, chunk_rows), :],
                outsem[slot],
            )

        for c in range(min(LA, n_chunks)):
            for cp in in_copies(c):
                cp.start()
        for c in range(n_chunks):
            slot = c % NSLOT
            j = c + LA
            if j < n_chunks:
                if j >= NSLOT:
                    out_copy(j - NSLOT).wait()
                for cp in in_copies(j):
                    cp.start()
            for cp in in_copies(c):
                cp.wait()
            xb = xbuf[slot]
            eb = ebuf[slot]

            @plsc.parallel_loop(0, chunk_rows, 1)
            def _(r):
                @plsc.parallel_loop(0, D, L, unroll=8)
                def _(k):
                    e = eb[r, pl.ds(k, L)]
                    for b in range(B):
                        xb[b, r, pl.ds(k, L)] = xb[b, r, pl.ds(k, L)] + e

            out_copy(c).start()
        for c in range(max(0, n_chunks - NSLOT), n_chunks):
            out_copy(c).wait()

    return sc_add


def kernel(x, emb):
    B, S, D = x.shape
    info = plsc.get_sparse_core_info()
    NC, NS, L = info.num_cores, info.num_subcores, info.num_lanes
    sc_add = _make_sc_add(B, S, D, NC, NS, L)
    return sc_add(x, emb[:S])
